# baseline (speedup 1.0000x reference)
import jax
import jax.numpy as jnp
from jax import lax
from jax.experimental import pallas as pl
from jax.experimental.pallas import tpu as pltpu

BATCH = 128
NF = 32768
RB = 8
TILES = NF // 128

_F32_BIG = 3.4e38


def _tc_body(x_ref, p_ref, o_ref):
    p = p_ref[...]

    def it(j, carry):
        mind, minv = carry
        x = x_ref[:, pl.ds(pl.multiple_of(j * 128, 128), 128)]
        d = jnp.abs(x - p)
        pred = d < mind
        return jnp.where(pred, d, mind), jnp.where(pred, x, minv)

    mind, minv = lax.fori_loop(
        0,
        TILES,
        it,
        (
            jnp.full((RB, 128), _F32_BIG, jnp.float32),
            jnp.zeros((RB, 128), jnp.float32),
        ),
    )
    m = jnp.min(mind, axis=1, keepdims=True)
    val = jnp.min(
        jnp.where(mind <= m, minv, jnp.float32(_F32_BIG)), axis=1, keepdims=True
    )
    o_ref[...] = val


@jax.jit
def _closest(inp, prev):
    return pl.pallas_call(
        _tc_body,
        grid=(BATCH // RB,),
        in_specs=[
            pl.BlockSpec((RB, NF), lambda i: (i, 0)),
            pl.BlockSpec((RB, 1), lambda i: (i, 0)),
        ],
        out_specs=pl.BlockSpec((RB, 1), lambda i: (i, 0)),
        out_shape=jax.ShapeDtypeStruct((BATCH, 1), jnp.float32),
    )(inp, prev)


def kernel(input, prev_output):
    return _closest(input, prev_output)


# speedup vs baseline: 14.6051x; 14.6051x over previous
import jax
import jax.numpy as jnp
from jax import lax
from jax.experimental import pallas as pl
from jax.experimental.pallas import tpu as pltpu

BATCH = 128
NF = 32768
RB = 8
TILES = NF // 128

_F32_BIG = 3.4e38


def _tc_body(x_ref, p_ref, o_ref):
    x = x_ref[...]
    d = jnp.abs(x - p_ref[...])
    m = jnp.min(d, axis=1, keepdims=True)
    val = jnp.min(
        jnp.where(d <= m, x, jnp.float32(_F32_BIG)), axis=1, keepdims=True
    )
    o_ref[...] = val


@jax.jit
def _closest(inp, prev):
    return pl.pallas_call(
        _tc_body,
        grid=(BATCH // RB,),
        in_specs=[
            pl.BlockSpec((RB, NF), lambda i: (i, 0)),
            pl.BlockSpec((RB, 1), lambda i: (i, 0)),
        ],
        out_specs=pl.BlockSpec((RB, 1), lambda i: (i, 0)),
        out_shape=jax.ShapeDtypeStruct((BATCH, 1), jnp.float32),
    )(inp, prev)


def kernel(input, prev_output):
    return _closest(input, prev_output)


# speedup vs baseline: 15.6313x; 1.0703x over previous
import jax
import jax.numpy as jnp
from jax import lax
from jax.experimental import pallas as pl
from jax.experimental.pallas import tpu as pltpu

BATCH = 128
NF = 32768
RB = 8
TILES = NF // 128

_F32_BIG = 3.4e38


def _tc_body(x_ref, p_ref, o_ref):
    p = p_ref[...]
    t = x_ref[...] - p
    ti = lax.bitcast_convert_type(t, jnp.int32)
    a = ti & jnp.int32(0x7FFFFFFF)
    s = lax.shift_right_logical(ti, 31)
    e = (a & jnp.int32(-2)) | s
    m = jnp.min(e, axis=1, keepdims=True)
    d_rec = lax.bitcast_convert_type(m & jnp.int32(-2), jnp.float32)
    val = p + jnp.where(m & 1, -d_rec, d_rec)
    o_ref[...] = val


@jax.jit
def _closest(inp, prev):
    return pl.pallas_call(
        _tc_body,
        grid=(BATCH // RB,),
        in_specs=[
            pl.BlockSpec((RB, NF), lambda i: (i, 0)),
            pl.BlockSpec((RB, 1), lambda i: (i, 0)),
        ],
        out_specs=pl.BlockSpec((RB, 1), lambda i: (i, 0)),
        out_shape=jax.ShapeDtypeStruct((BATCH, 1), jnp.float32),
    )(inp, prev)


def kernel(input, prev_output):
    return _closest(input, prev_output)


# speedup vs baseline: 16.6819x; 1.0672x over previous
import jax
import jax.numpy as jnp
from jax import lax
from jax.experimental import pallas as pl
from jax.experimental.pallas import tpu as pltpu

BATCH = 128
NF = 32768
RB = 8
TILES = NF // 128

_F32_BIG = 3.4e38


_NACC = 4
_TW = 512


def _tc_body(x_ref, p_ref, o_ref):
    p = p_ref[...]
    accs = [jnp.full((RB, _TW), 0x7F800000, jnp.int32) for _ in range(_NACC)]
    for j in range(NF // _TW):
        t = x_ref[:, j * _TW:(j + 1) * _TW] - p
        ti = lax.bitcast_convert_type(t, jnp.int32)
        a = ti & jnp.int32(0x7FFFFFFF)
        s = lax.shift_right_logical(ti, 31)
        e = (a & jnp.int32(-2)) | s
        accs[j % _NACC] = jnp.minimum(accs[j % _NACC], e)
    acc = jnp.minimum(
        jnp.minimum(accs[0], accs[1]), jnp.minimum(accs[2], accs[3])
    )
    m = jnp.min(acc, axis=1, keepdims=True)
    d_rec = lax.bitcast_convert_type(m & jnp.int32(-2), jnp.float32)
    val = p + jnp.where(m & 1, -d_rec, d_rec)
    o_ref[...] = val


@jax.jit
def _closest(inp, prev):
    return pl.pallas_call(
        _tc_body,
        grid=(BATCH // RB,),
        in_specs=[
            pl.BlockSpec((RB, NF), lambda i: (i, 0)),
            pl.BlockSpec((RB, 1), lambda i: (i, 0)),
        ],
        out_specs=pl.BlockSpec((RB, 1), lambda i: (i, 0)),
        out_shape=jax.ShapeDtypeStruct((BATCH, 1), jnp.float32),
    )(inp, prev)


def kernel(input, prev_output):
    return _closest(input, prev_output)


# speedup vs baseline: 22.8383x; 1.3690x over previous
import jax
import jax.numpy as jnp
from jax import lax
from jax.experimental import pallas as pl
from jax.experimental.pallas import tpu as pltpu

BATCH = 128
NF = 32768
RB = 16
TILES = NF // 128

_F32_BIG = 3.4e38


_NACC = 4
_TW = 512


def _tc_body(x_ref, p_ref, o_ref):
    p = p_ref[...]
    accs = [
        jnp.full((RB, _TW), _F32_BIG, jnp.float32) for _ in range(_NACC)
    ]
    for j in range(NF // _TW):
        t = x_ref[:, j * _TW:(j + 1) * _TW] - p
        ti = lax.bitcast_convert_type(t, jnp.int32)
        a = ti & jnp.int32(0x7FFFFFFE)
        s = lax.shift_right_logical(ti, 31)
        e = lax.bitcast_convert_type(a | s, jnp.float32)
        accs[j % _NACC] = jnp.minimum(accs[j % _NACC], e)
    acc = jnp.minimum(
        jnp.minimum(accs[0], accs[1]), jnp.minimum(accs[2], accs[3])
    )
    m = jnp.min(acc, axis=1, keepdims=True)
    mi = lax.bitcast_convert_type(m, jnp.int32)
    d_rec = lax.bitcast_convert_type(mi & jnp.int32(-2), jnp.float32)
    val = p + jnp.where(mi & 1, -d_rec, d_rec)
    o_ref[...] = val


@jax.jit
def _closest(inp, prev):
    return pl.pallas_call(
        _tc_body,
        grid=(BATCH // RB,),
        in_specs=[
            pl.BlockSpec((RB, NF), lambda i: (i, 0)),
            pl.BlockSpec((RB, 1), lambda i: (i, 0)),
        ],
        out_specs=pl.BlockSpec((RB, 1), lambda i: (i, 0)),
        out_shape=jax.ShapeDtypeStruct((BATCH, 1), jnp.float32),
    )(inp, prev)


def kernel(input, prev_output):
    return _closest(input, prev_output)


# speedup vs baseline: 26.9302x; 1.1792x over previous
import jax
import jax.numpy as jnp
from jax import lax
from jax.experimental import pallas as pl
from jax.experimental.pallas import tpu as pltpu

BATCH = 128
NF = 32768
RB = 32
TILES = NF // 128

_F32_BIG = 3.4e38


_NACC = 4
_TW = 512


def _tc_body(x_ref, p_ref, o_ref):
    p = p_ref[...]
    accs = [
        jnp.full((RB, _TW), _F32_BIG, jnp.float32) for _ in range(_NACC)
    ]
    for j in range(NF // _TW):
        t = x_ref[:, j * _TW:(j + 1) * _TW] - p
        ti = lax.bitcast_convert_type(t, jnp.int32)
        a = ti & jnp.int32(0x7FFFFFFE)
        s = lax.shift_right_logical(ti, 31)
        e = lax.bitcast_convert_type(a | s, jnp.float32)
        accs[j % _NACC] = jnp.minimum(accs[j % _NACC], e)
    acc = jnp.minimum(
        jnp.minimum(accs[0], accs[1]), jnp.minimum(accs[2], accs[3])
    )
    m = jnp.min(acc, axis=1, keepdims=True)
    mi = lax.bitcast_convert_type(m, jnp.int32)
    d_rec = lax.bitcast_convert_type(mi & jnp.int32(-2), jnp.float32)
    val = p + jnp.where(mi & 1, -d_rec, d_rec)
    o_ref[...] = val


@jax.jit
def _closest(inp, prev):
    return pl.pallas_call(
        _tc_body,
        grid=(BATCH // RB,),
        in_specs=[
            pl.BlockSpec((RB, NF), lambda i: (i, 0)),
            pl.BlockSpec((RB, 1), lambda i: (i, 0)),
        ],
        out_specs=pl.BlockSpec((RB, 1), lambda i: (i, 0)),
        out_shape=jax.ShapeDtypeStruct((BATCH, 1), jnp.float32),
    )(inp, prev)


def kernel(input, prev_output):
    return _closest(input, prev_output)


# speedup vs baseline: 27.8284x; 1.0334x over previous
import jax
import jax.numpy as jnp
from jax import lax
from jax.experimental import pallas as pl
from jax.experimental.pallas import tpu as pltpu

BATCH = 128
NF = 32768
RB = 64
TILES = NF // 128

_F32_BIG = 3.4e38


_NACC = 4
_TW = 512


def _tc_body(x_ref, p_ref, o_ref):
    p = p_ref[...]
    accs = [
        jnp.full((RB, _TW), _F32_BIG, jnp.float32) for _ in range(_NACC)
    ]
    for j in range(NF // _TW):
        t = x_ref[:, j * _TW:(j + 1) * _TW] - p
        ti = lax.bitcast_convert_type(t, jnp.int32)
        a = ti & jnp.int32(0x7FFFFFFE)
        s = lax.shift_right_logical(ti, 31)
        e = lax.bitcast_convert_type(a | s, jnp.float32)
        accs[j % _NACC] = jnp.minimum(accs[j % _NACC], e)
    acc = jnp.minimum(
        jnp.minimum(accs[0], accs[1]), jnp.minimum(accs[2], accs[3])
    )
    m = jnp.min(acc, axis=1, keepdims=True)
    mi = lax.bitcast_convert_type(m, jnp.int32)
    d_rec = lax.bitcast_convert_type(mi & jnp.int32(-2), jnp.float32)
    val = p + jnp.where(mi & 1, -d_rec, d_rec)
    o_ref[...] = val


@jax.jit
def _closest(inp, prev):
    return pl.pallas_call(
        _tc_body,
        grid=(BATCH // RB,),
        in_specs=[
            pl.BlockSpec((RB, NF), lambda i: (i, 0)),
            pl.BlockSpec((RB, 1), lambda i: (i, 0)),
        ],
        out_specs=pl.BlockSpec((RB, 1), lambda i: (i, 0)),
        out_shape=jax.ShapeDtypeStruct((BATCH, 1), jnp.float32),
    )(inp, prev)


def kernel(input, prev_output):
    return _closest(input, prev_output)
